# baseline (speedup 1.0000x reference)
import functools

import jax
import jax.numpy as jnp
from jax import lax
from jax.experimental import pallas as pl
from jax.experimental.pallas import tpu as pltpu
from jax.experimental.pallas import tpu_sc as plsc

N = 10000
E = 320000
D = 128
NC = 2
NS = 16
NW = NC * NS
CHUNK = 128
CH = 80
EPW = CH * CHUNK
E_PAD = NW * EPW
N_PAD = 10112
RPT = N_PAD // NS

_mesh = plsc.VectorSubcoreMesh(core_axis_name="c", subcore_axis_name="s")


@functools.partial(
    pl.kernel,
    out_type=jax.ShapeDtypeStruct((NC, N_PAD, D), jnp.float32),
    mesh=_mesh,
    scratch_types=[
        pltpu.VMEM((CH, CHUNK), jnp.int32),
        pltpu.VMEM((CHUNK, D), jnp.float32),
        pltpu.VMEM_SHARED((N_PAD, D), jnp.float32),
    ],
)
def _deg_kernel(dst_hbm, zeros_hbm, ones_hbm, out_hbm, dst_v, buf, acc_sh):
    c = lax.axis_index("c")
    s = lax.axis_index("s")
    wid = s * NC + c
    r0 = s * RPT
    pltpu.sync_copy(dst_hbm.at[pl.ds(wid * CH, CH)], dst_v)
    pltpu.sync_copy(zeros_hbm, buf)
    for k in range(4):
        pltpu.sync_copy(buf, acc_sh.at[pl.ds(r0 + k * CHUNK, CHUNK)])
    pltpu.sync_copy(buf.at[pl.ds(0, RPT - 4 * CHUNK)],
                    acc_sh.at[pl.ds(r0 + 4 * CHUNK, RPT - 4 * CHUNK)])
    pltpu.sync_copy(ones_hbm, buf)
    plsc.subcore_barrier()

    def body(j, carry):
        pltpu.sync_copy(buf, acc_sh.at[dst_v.at[j]], add=True)
        return carry

    lax.fori_loop(0, CH, body, 0)
    plsc.subcore_barrier()
    pltpu.sync_copy(acc_sh.at[pl.ds(r0, RPT)], out_hbm.at[c, pl.ds(r0, RPT)])


NCHUNKS = E_PAD // CHUNK
SLOW_C = 1
CS_G = 16
CF_G = 64
FAST_BASE = 2 * CS_G * NS


@functools.partial(
    pl.kernel,
    out_type=jax.ShapeDtypeStruct((NC, N_PAD, D), jnp.float32),
    mesh=_mesh,
    scratch_types=[
        pltpu.VMEM((CF_G, CHUNK), jnp.int32),
        pltpu.VMEM((CF_G, CHUNK), jnp.int32),
        pltpu.VMEM((CHUNK, D), jnp.float32),
        pltpu.VMEM((CHUNK, D), jnp.float32),
        pltpu.VMEM_SHARED((N_PAD, D), jnp.float32),
        pltpu.SemaphoreType.DMA,
        pltpu.SemaphoreType.DMA,
    ],
)
def _agg_kernel(g_hbm, src_hbm, dst_hbm, zeros_hbm, out_hbm, src_v, dst_v,
                buf0, buf1, acc_sh, sem0, sem1):
    c = lax.axis_index("c")
    s = lax.axis_index("s")
    r0 = s * RPT
    slow = c == SLOW_C
    n = jnp.where(slow, CS_G, CF_G)
    pltpu.sync_copy(zeros_hbm, buf0)
    for k in range(4):
        pltpu.sync_copy(buf0, acc_sh.at[pl.ds(r0 + k * CHUNK, CHUNK)])
    pltpu.sync_copy(buf0.at[pl.ds(0, RPT - 4 * CHUNK)],
                    acc_sh.at[pl.ds(r0 + 4 * CHUNK, RPT - 4 * CHUNK)])
    plsc.subcore_barrier()

    for g in range(2):
        base = jnp.where(slow, s * 2 * CS_G + g * CS_G,
                         FAST_BASE + s * 2 * CF_G + g * CF_G)
        pltpu.sync_copy(src_hbm.at[pl.ds(base, CF_G)], src_v)
        pltpu.sync_copy(dst_hbm.at[pl.ds(base, CF_G)], dst_v)
        pltpu.async_copy(g_hbm.at[src_v.at[0]], buf0, sem0)
        pltpu.async_copy(g_hbm.at[src_v.at[1]], buf1, sem1)

        def body(j2, carry):
            c0 = 2 * j2
            pltpu.make_async_copy(g_hbm.at[src_v.at[c0]], buf0, sem0).wait()
            pltpu.sync_copy(buf0, acc_sh.at[dst_v.at[c0]], add=True)
            pltpu.async_copy(g_hbm.at[src_v.at[c0 + 2]], buf0, sem0)
            pltpu.make_async_copy(g_hbm.at[src_v.at[c0 + 1]], buf1, sem1).wait()
            pltpu.sync_copy(buf1, acc_sh.at[dst_v.at[c0 + 1]], add=True)
            pltpu.async_copy(g_hbm.at[src_v.at[c0 + 3]], buf1, sem1)
            return carry

        lax.fori_loop(0, n // 2 - 1, body, 0)
        pltpu.make_async_copy(g_hbm.at[src_v.at[n - 2]], buf0, sem0).wait()
        pltpu.sync_copy(buf0, acc_sh.at[dst_v.at[n - 2]], add=True)
        pltpu.make_async_copy(g_hbm.at[src_v.at[n - 1]], buf1, sem1).wait()
        pltpu.sync_copy(buf1, acc_sh.at[dst_v.at[n - 1]], add=True)
    plsc.subcore_barrier()
    pltpu.sync_copy(acc_sh.at[pl.ds(r0, RPT)], out_hbm.at[c, pl.ds(r0, RPT)])


R_BLK = 1000
GRID = N // R_BLK


def _dinv_from_parts(degp):
    deg = degp[0][:, 0:1] + degp[1][:, 0:1] + 1.0
    return lax.rsqrt(deg)


def _first_body(x_ref, w_ref, degp_ref, g_ref):
    dinv = _dinv_from_parts(degp_ref[...])
    h = jnp.dot(x_ref[...], w_ref[...], preferred_element_type=jnp.float32)
    g_ref[...] = h * dinv


def _mid_body(p_ref, g_ref, degp_ref, b_ref, w_ref, out_ref):
    dinv = _dinv_from_parts(degp_ref[...])
    agg = p_ref[0] + p_ref[1] + g_ref[...]
    z = jnp.maximum(agg * dinv + b_ref[...], 0.0)
    out_ref[...] = jnp.dot(z, w_ref[...], preferred_element_type=jnp.float32) * dinv


def _last_body(p_ref, g_ref, degp_ref, b_ref, out_ref):
    dinv = _dinv_from_parts(degp_ref[...])
    agg = p_ref[0] + p_ref[1] + g_ref[...]
    out_ref[...] = agg * dinv + b_ref[...]


_row_spec = pl.BlockSpec((R_BLK, D), lambda i: (i, 0))
_w_spec = pl.BlockSpec((D, D), lambda i: (0, 0))
_b_spec = pl.BlockSpec((1, D), lambda i: (0, 0))
_degp_spec = pl.BlockSpec((NC, R_BLK, D), lambda i: (0, i, 0))
_part_spec = pl.BlockSpec((NC, R_BLK, D), lambda i: (0, i, 0))
_row_out = jax.ShapeDtypeStruct((N, D), jnp.float32)

_first_tc = pl.pallas_call(
    _first_body, grid=(GRID,),
    in_specs=[_row_spec, _w_spec, _degp_spec],
    out_specs=_row_spec, out_shape=_row_out)

_mid_tc = pl.pallas_call(
    _mid_body, grid=(GRID,),
    in_specs=[_part_spec, _row_spec, _degp_spec, _b_spec, _w_spec],
    out_specs=_row_spec, out_shape=_row_out)

_last_tc = pl.pallas_call(
    _last_body, grid=(GRID,),
    in_specs=[_part_spec, _row_spec, _degp_spec, _b_spec],
    out_specs=_row_spec, out_shape=_row_out)


def kernel(x, edge_index, W1, b1, W2, b2, W3, b3):
    src = edge_index[0]
    dst = edge_index[1]
    pad = E_PAD - E
    srcp = jnp.concatenate([src, jnp.zeros((pad,), jnp.int32)])
    dstp = jnp.concatenate(
        [dst, N + (jnp.arange(pad, dtype=jnp.int32) % NS)])
    srcp = srcp.reshape(E_PAD // CHUNK, CHUNK)
    dstp = dstp.reshape(E_PAD // CHUNK, CHUNK)
    ones_row = jnp.ones((CHUNK, D), jnp.float32)
    zrow = jnp.zeros((CHUNK, D), jnp.float32)
    b1r = b1.reshape(1, D)
    b2r = b2.reshape(1, D)
    b3r = b3.reshape(1, D)

    degp = _deg_kernel(dstp, zrow, ones_row)
    g1 = _first_tc(x, W1, degp)
    p1 = _agg_kernel(g1, srcp, dstp, zrow)
    g2 = _mid_tc(p1, g1, degp, b1r, W2)
    p2 = _agg_kernel(g2, srcp, dstp, zrow)
    g3 = _mid_tc(p2, g2, degp, b2r, W3)
    p3 = _agg_kernel(g3, srcp, dstp, zrow)
    return _last_tc(p3, g3, degp, b3r)


# speedup vs baseline: 1.0025x; 1.0025x over previous
import functools

import jax
import jax.numpy as jnp
from jax import lax
from jax.experimental import pallas as pl
from jax.experimental.pallas import tpu as pltpu
from jax.experimental.pallas import tpu_sc as plsc

N = 10000
E = 320000
D = 128
NC = 2
NS = 16
NW = NC * NS
CHUNK = 128
CH = 80
EPW = CH * CHUNK
E_PAD = NW * EPW
N_PAD = 10112
RPT = N_PAD // NS

_mesh = plsc.VectorSubcoreMesh(core_axis_name="c", subcore_axis_name="s")


@functools.partial(
    pl.kernel,
    out_type=jax.ShapeDtypeStruct((NC, N_PAD, D), jnp.float32),
    mesh=_mesh,
    scratch_types=[
        pltpu.VMEM((CH, CHUNK), jnp.int32),
        pltpu.VMEM((CHUNK, D), jnp.float32),
        pltpu.VMEM_SHARED((N_PAD, D), jnp.float32),
    ],
)
def _deg_kernel(dst_hbm, zeros_hbm, ones_hbm, out_hbm, dst_v, buf, acc_sh):
    c = lax.axis_index("c")
    s = lax.axis_index("s")
    wid = s * NC + c
    r0 = s * RPT
    pltpu.sync_copy(dst_hbm.at[wid], dst_v)
    pltpu.sync_copy(zeros_hbm, buf)
    for k in range(4):
        pltpu.sync_copy(buf, acc_sh.at[pl.ds(r0 + k * CHUNK, CHUNK)])
    pltpu.sync_copy(buf.at[pl.ds(0, RPT - 4 * CHUNK)],
                    acc_sh.at[pl.ds(r0 + 4 * CHUNK, RPT - 4 * CHUNK)])
    pltpu.sync_copy(ones_hbm, buf)
    plsc.subcore_barrier()

    def body(j, carry):
        pltpu.sync_copy(buf, acc_sh.at[dst_v.at[j]], add=True)
        return carry

    lax.fori_loop(0, CH, body, 0)
    plsc.subcore_barrier()
    pltpu.sync_copy(acc_sh.at[pl.ds(r0, RPT)], out_hbm.at[c, pl.ds(r0, RPT)])


CH_G = 40
NG = CH // CH_G


@functools.partial(
    pl.kernel,
    out_type=jax.ShapeDtypeStruct((NC, N_PAD, D), jnp.float32),
    mesh=_mesh,
    scratch_types=[
        pltpu.VMEM((CH_G, CHUNK), jnp.int32),
        pltpu.VMEM((CH_G, CHUNK), jnp.int32),
        pltpu.VMEM((CHUNK, D), jnp.float32),
        pltpu.VMEM((CHUNK, D), jnp.float32),
        pltpu.VMEM_SHARED((N_PAD, D), jnp.float32),
        pltpu.SemaphoreType.DMA,
        pltpu.SemaphoreType.DMA,
        pltpu.SemaphoreType.DMA,
        pltpu.SemaphoreType.DMA,
    ],
)
def _agg_kernel(g_hbm, src_hbm, dst_hbm, zeros_hbm, out_hbm, src_v, dst_v,
                buf0, buf1, acc_sh, sem0, sem1, ss0, ss1):
    c = lax.axis_index("c")
    s = lax.axis_index("s")
    wid = s * NC + c
    r0 = s * RPT
    pltpu.sync_copy(zeros_hbm, buf0)
    for k in range(4):
        pltpu.sync_copy(buf0, acc_sh.at[pl.ds(r0 + k * CHUNK, CHUNK)])
    pltpu.sync_copy(buf0.at[pl.ds(0, RPT - 4 * CHUNK)],
                    acc_sh.at[pl.ds(r0 + 4 * CHUNK, RPT - 4 * CHUNK)])
    plsc.subcore_barrier()

    for g in range(NG):
        pltpu.sync_copy(src_hbm.at[wid, pl.ds(g * CH_G, CH_G)], src_v)
        pltpu.sync_copy(dst_hbm.at[wid, pl.ds(g * CH_G, CH_G)], dst_v)
        pltpu.async_copy(g_hbm.at[src_v.at[0]], buf0, sem0)
        pltpu.async_copy(g_hbm.at[src_v.at[1]], buf1, sem1)

        def body(j2, carry):
            c0 = 2 * j2
            pltpu.make_async_copy(g_hbm.at[src_v.at[c0]], buf0, sem0).wait()
            pltpu.async_copy(buf0, acc_sh.at[dst_v.at[c0]], ss0, add=True)
            pltpu.make_async_copy(g_hbm.at[src_v.at[c0 + 1]], buf1, sem1).wait()
            pltpu.async_copy(buf1, acc_sh.at[dst_v.at[c0 + 1]], ss1, add=True)
            pltpu.make_async_copy(buf0, acc_sh.at[dst_v.at[c0]], ss0).wait()
            pltpu.async_copy(g_hbm.at[src_v.at[c0 + 2]], buf0, sem0)
            pltpu.make_async_copy(buf1, acc_sh.at[dst_v.at[c0 + 1]], ss1).wait()
            pltpu.async_copy(g_hbm.at[src_v.at[c0 + 3]], buf1, sem1)
            return carry

        lax.fori_loop(0, CH_G // 2 - 1, body, 0)
        pltpu.make_async_copy(g_hbm.at[src_v.at[CH_G - 2]], buf0, sem0).wait()
        pltpu.sync_copy(buf0, acc_sh.at[dst_v.at[CH_G - 2]], add=True)
        pltpu.make_async_copy(g_hbm.at[src_v.at[CH_G - 1]], buf1, sem1).wait()
        pltpu.sync_copy(buf1, acc_sh.at[dst_v.at[CH_G - 1]], add=True)
    plsc.subcore_barrier()
    pltpu.sync_copy(acc_sh.at[pl.ds(r0, RPT)], out_hbm.at[c, pl.ds(r0, RPT)])


R_BLK = 1000
GRID = N // R_BLK


def _dinv_from_parts(degp):
    deg = degp[0][:, 0:1] + degp[1][:, 0:1] + 1.0
    return lax.rsqrt(deg)


def _first_body(x_ref, w_ref, degp_ref, g_ref):
    dinv = _dinv_from_parts(degp_ref[...])
    h = jnp.dot(x_ref[...], w_ref[...], preferred_element_type=jnp.float32)
    g_ref[...] = h * dinv


def _mid_body(p_ref, g_ref, degp_ref, b_ref, w_ref, out_ref):
    dinv = _dinv_from_parts(degp_ref[...])
    agg = p_ref[0] + p_ref[1] + g_ref[...]
    z = jnp.maximum(agg * dinv + b_ref[...], 0.0)
    out_ref[...] = jnp.dot(z, w_ref[...], preferred_element_type=jnp.float32) * dinv


def _last_body(p_ref, g_ref, degp_ref, b_ref, out_ref):
    dinv = _dinv_from_parts(degp_ref[...])
    agg = p_ref[0] + p_ref[1] + g_ref[...]
    out_ref[...] = agg * dinv + b_ref[...]


_row_spec = pl.BlockSpec((R_BLK, D), lambda i: (i, 0))
_w_spec = pl.BlockSpec((D, D), lambda i: (0, 0))
_b_spec = pl.BlockSpec((1, D), lambda i: (0, 0))
_degp_spec = pl.BlockSpec((NC, R_BLK, D), lambda i: (0, i, 0))
_part_spec = pl.BlockSpec((NC, R_BLK, D), lambda i: (0, i, 0))
_row_out = jax.ShapeDtypeStruct((N, D), jnp.float32)

_first_tc = pl.pallas_call(
    _first_body, grid=(GRID,),
    in_specs=[_row_spec, _w_spec, _degp_spec],
    out_specs=_row_spec, out_shape=_row_out)

_mid_tc = pl.pallas_call(
    _mid_body, grid=(GRID,),
    in_specs=[_part_spec, _row_spec, _degp_spec, _b_spec, _w_spec],
    out_specs=_row_spec, out_shape=_row_out)

_last_tc = pl.pallas_call(
    _last_body, grid=(GRID,),
    in_specs=[_part_spec, _row_spec, _degp_spec, _b_spec],
    out_specs=_row_spec, out_shape=_row_out)


def kernel(x, edge_index, W1, b1, W2, b2, W3, b3):
    src = edge_index[0]
    dst = edge_index[1]
    pad = E_PAD - E
    srcp = jnp.concatenate([src, jnp.zeros((pad,), jnp.int32)])
    dstp = jnp.concatenate(
        [dst, N + (jnp.arange(pad, dtype=jnp.int32) % NS)])
    srcp = srcp.reshape(NW, CH, CHUNK)
    dstp = dstp.reshape(NW, CH, CHUNK)
    ones_row = jnp.ones((CHUNK, D), jnp.float32)
    zrow = jnp.zeros((CHUNK, D), jnp.float32)
    b1r = b1.reshape(1, D)
    b2r = b2.reshape(1, D)
    b3r = b3.reshape(1, D)

    degp = _deg_kernel(dstp, zrow, ones_row)
    g1 = _first_tc(x, W1, degp)
    p1 = _agg_kernel(g1, srcp, dstp, zrow)
    g2 = _mid_tc(p1, g1, degp, b1r, W2)
    p2 = _agg_kernel(g2, srcp, dstp, zrow)
    g3 = _mid_tc(p2, g2, degp, b2r, W3)
    p3 = _agg_kernel(g3, srcp, dstp, zrow)
    return _last_tc(p3, g3, degp, b3r)


# speedup vs baseline: 1.0562x; 1.0535x over previous
import functools

import jax
import jax.numpy as jnp
from jax import lax
from jax.experimental import pallas as pl
from jax.experimental.pallas import tpu as pltpu
from jax.experimental.pallas import tpu_sc as plsc

N = 10000
E = 320000
D = 128
NC = 2
NS = 16
NW = NC * NS
CHUNK = 128
CH = 80
EPW = CH * CHUNK
E_PAD = NW * EPW
N_PAD = 10112
RPT = N_PAD // NS

_mesh = plsc.VectorSubcoreMesh(core_axis_name="c", subcore_axis_name="s")


@functools.partial(
    pl.kernel,
    out_type=jax.ShapeDtypeStruct((NC, N_PAD, D), jnp.float32),
    mesh=_mesh,
    scratch_types=[
        pltpu.VMEM((CH, CHUNK), jnp.int32),
        pltpu.VMEM((CHUNK, D), jnp.float32),
        pltpu.VMEM_SHARED((N_PAD, D), jnp.float32),
    ],
)
def _deg_kernel(dst_hbm, zeros_hbm, ones_hbm, out_hbm, dst_v, buf, acc_sh):
    c = lax.axis_index("c")
    s = lax.axis_index("s")
    wid = s * NC + c
    r0 = s * RPT
    pltpu.sync_copy(dst_hbm.at[wid], dst_v)
    pltpu.sync_copy(zeros_hbm, buf)
    for k in range(4):
        pltpu.sync_copy(buf, acc_sh.at[pl.ds(r0 + k * CHUNK, CHUNK)])
    pltpu.sync_copy(buf.at[pl.ds(0, RPT - 4 * CHUNK)],
                    acc_sh.at[pl.ds(r0 + 4 * CHUNK, RPT - 4 * CHUNK)])
    pltpu.sync_copy(ones_hbm, buf)
    plsc.subcore_barrier()

    def body(j, carry):
        pltpu.sync_copy(buf, acc_sh.at[dst_v.at[j]], add=True)
        return carry

    lax.fori_loop(0, CH, body, 0)
    plsc.subcore_barrier()
    pltpu.sync_copy(acc_sh.at[pl.ds(r0, RPT)], out_hbm.at[c, pl.ds(r0, RPT)])


CH_G = 40
NG = CH // CH_G


@functools.partial(
    pl.kernel,
    out_type=jax.ShapeDtypeStruct((NC, N_PAD, D), jnp.float32),
    mesh=_mesh,
    scratch_types=[
        pltpu.VMEM((CH_G, CHUNK), jnp.int32),
        pltpu.VMEM((CH_G, CHUNK), jnp.int32),
        pltpu.VMEM((CHUNK, D), jnp.float32),
        pltpu.VMEM((CHUNK, D), jnp.float32),
        pltpu.VMEM_SHARED((N_PAD, D), jnp.float32),
        pltpu.SemaphoreType.DMA,
        pltpu.SemaphoreType.DMA,
    ],
)
def _agg_kernel(g_hbm, src_hbm, dst_hbm, zeros_hbm, out_hbm, src_v, dst_v,
                buf0, buf1, acc_sh, sem0, sem1):
    c = lax.axis_index("c")
    s = lax.axis_index("s")
    wid = s * NC + c
    r0 = s * RPT
    pltpu.sync_copy(zeros_hbm, buf0)
    for k in range(4):
        pltpu.sync_copy(buf0, acc_sh.at[pl.ds(r0 + k * CHUNK, CHUNK)])
    pltpu.sync_copy(buf0.at[pl.ds(0, RPT - 4 * CHUNK)],
                    acc_sh.at[pl.ds(r0 + 4 * CHUNK, RPT - 4 * CHUNK)])
    plsc.subcore_barrier()

    def issue_gather(c, buf, sem):
        pltpu.async_copy(g_hbm.at[src_v.at[c, pl.ds(0, 64)]],
                         buf.at[pl.ds(0, 64)], sem)
        pltpu.async_copy(g_hbm.at[src_v.at[c, pl.ds(64, 64)]],
                         buf.at[pl.ds(64, 64)], sem)

    def wait_gather(c, buf, sem):
        pltpu.make_async_copy(g_hbm.at[src_v.at[c, pl.ds(0, 64)]],
                              buf.at[pl.ds(0, 64)], sem).wait()
        pltpu.make_async_copy(g_hbm.at[src_v.at[c, pl.ds(64, 64)]],
                              buf.at[pl.ds(64, 64)], sem).wait()

    for g in range(NG):
        pltpu.sync_copy(src_hbm.at[wid, pl.ds(g * CH_G, CH_G)], src_v)
        pltpu.sync_copy(dst_hbm.at[wid, pl.ds(g * CH_G, CH_G)], dst_v)
        issue_gather(0, buf0, sem0)
        issue_gather(1, buf1, sem1)

        def body(j2, carry):
            c0 = 2 * j2
            wait_gather(c0, buf0, sem0)
            pltpu.sync_copy(buf0, acc_sh.at[dst_v.at[c0]], add=True)
            issue_gather(c0 + 2, buf0, sem0)
            wait_gather(c0 + 1, buf1, sem1)
            pltpu.sync_copy(buf1, acc_sh.at[dst_v.at[c0 + 1]], add=True)
            issue_gather(c0 + 3, buf1, sem1)
            return carry

        lax.fori_loop(0, CH_G // 2 - 1, body, 0)
        wait_gather(CH_G - 2, buf0, sem0)
        pltpu.sync_copy(buf0, acc_sh.at[dst_v.at[CH_G - 2]], add=True)
        wait_gather(CH_G - 1, buf1, sem1)
        pltpu.sync_copy(buf1, acc_sh.at[dst_v.at[CH_G - 1]], add=True)
    plsc.subcore_barrier()
    pltpu.sync_copy(acc_sh.at[pl.ds(r0, RPT)], out_hbm.at[c, pl.ds(r0, RPT)])


R_BLK = 1000
GRID = N // R_BLK


def _dinv_from_parts(degp):
    deg = degp[0][:, 0:1] + degp[1][:, 0:1] + 1.0
    return lax.rsqrt(deg)


def _first_body(x_ref, w_ref, degp_ref, g_ref):
    dinv = _dinv_from_parts(degp_ref[...])
    h = jnp.dot(x_ref[...], w_ref[...], preferred_element_type=jnp.float32)
    g_ref[...] = h * dinv


def _mid_body(p_ref, g_ref, degp_ref, b_ref, w_ref, out_ref):
    dinv = _dinv_from_parts(degp_ref[...])
    agg = p_ref[0] + p_ref[1] + g_ref[...]
    z = jnp.maximum(agg * dinv + b_ref[...], 0.0)
    out_ref[...] = jnp.dot(z, w_ref[...], preferred_element_type=jnp.float32) * dinv


def _last_body(p_ref, g_ref, degp_ref, b_ref, out_ref):
    dinv = _dinv_from_parts(degp_ref[...])
    agg = p_ref[0] + p_ref[1] + g_ref[...]
    out_ref[...] = agg * dinv + b_ref[...]


_row_spec = pl.BlockSpec((R_BLK, D), lambda i: (i, 0))
_w_spec = pl.BlockSpec((D, D), lambda i: (0, 0))
_b_spec = pl.BlockSpec((1, D), lambda i: (0, 0))
_degp_spec = pl.BlockSpec((NC, R_BLK, D), lambda i: (0, i, 0))
_part_spec = pl.BlockSpec((NC, R_BLK, D), lambda i: (0, i, 0))
_row_out = jax.ShapeDtypeStruct((N, D), jnp.float32)

_first_tc = pl.pallas_call(
    _first_body, grid=(GRID,),
    in_specs=[_row_spec, _w_spec, _degp_spec],
    out_specs=_row_spec, out_shape=_row_out)

_mid_tc = pl.pallas_call(
    _mid_body, grid=(GRID,),
    in_specs=[_part_spec, _row_spec, _degp_spec, _b_spec, _w_spec],
    out_specs=_row_spec, out_shape=_row_out)

_last_tc = pl.pallas_call(
    _last_body, grid=(GRID,),
    in_specs=[_part_spec, _row_spec, _degp_spec, _b_spec],
    out_specs=_row_spec, out_shape=_row_out)


def kernel(x, edge_index, W1, b1, W2, b2, W3, b3):
    src = edge_index[0]
    dst = edge_index[1]
    pad = E_PAD - E
    srcp = jnp.concatenate([src, jnp.zeros((pad,), jnp.int32)])
    dstp = jnp.concatenate(
        [dst, N + (jnp.arange(pad, dtype=jnp.int32) % NS)])
    srcp = srcp.reshape(NW, CH, CHUNK)
    dstp = dstp.reshape(NW, CH, CHUNK)
    ones_row = jnp.ones((CHUNK, D), jnp.float32)
    zrow = jnp.zeros((CHUNK, D), jnp.float32)
    b1r = b1.reshape(1, D)
    b2r = b2.reshape(1, D)
    b3r = b3.reshape(1, D)

    degp = _deg_kernel(dstp, zrow, ones_row)
    g1 = _first_tc(x, W1, degp)
    p1 = _agg_kernel(g1, srcp, dstp, zrow)
    g2 = _mid_tc(p1, g1, degp, b1r, W2)
    p2 = _agg_kernel(g2, srcp, dstp, zrow)
    g3 = _mid_tc(p2, g2, degp, b2r, W3)
    p3 = _agg_kernel(g3, srcp, dstp, zrow)
    return _last_tc(p3, g3, degp, b3r)
